# baseline (speedup 1.0000x reference)
import functools

import jax
import jax.numpy as jnp
from jax.experimental import pallas as pl
from jax.experimental.pallas import tpu as pltpu

N_LEFT = 8
N_RIGHT = 8
N_HEADS = 12
_BT = 256
_CH = 128
_KC = _CH + 16
_NEG = -1e9
_CSHIFT = -30.0


def _attn_body(e_ref, isn_ref, w3_ref, wo_ref, o_ref, *, T, D):
    H = N_HEADS
    dh = D // H
    i = pl.program_id(1)
    t0 = i * _BT

    ec = e_ref[0, pl.ds(t0 + N_LEFT, _BT), :]
    eh = e_ref[0, pl.ds(t0, _BT + 16), :].astype(jnp.bfloat16)
    q = jnp.dot(ec.astype(jnp.bfloat16), w3_ref[:, 0:D],
                preferred_element_type=jnp.float32).astype(jnp.bfloat16)
    kv = jnp.dot(eh, w3_ref[:, D:3 * D],
                 preferred_element_type=jnp.float32).astype(jnp.bfloat16)

    qi = jax.lax.broadcasted_iota(jnp.int32, (_CH, _KC), 0)
    kj = jax.lax.broadcasted_iota(jnp.int32, (_CH, _KC), 1)
    band = (kj >= qi) & (kj <= qi + N_LEFT + N_RIGHT)

    rows = []
    for c in range(_BT // _CH):
        base = c * _CH
        q_c = q[base:base + _CH, :]
        k_c = kv[base:base + _KC, 0:D]
        v_c = kv[base:base + _KC, D:2 * D]
        pos_k = t0 + base + kj - N_LEFT
        mask = band & (pos_k >= 0) & (pos_k < T)
        bias = jnp.where(mask, jnp.float32(_CSHIFT), jnp.float32(_NEG))
        outs = []
        for h in range(H):
            sl = slice(h * dh, (h + 1) * dh)
            s = jax.lax.dot_general(q_c[:, sl], k_c[:, sl],
                                    (((1,), (1,)), ((), ())),
                                    preferred_element_type=jnp.float32)
            p = jnp.exp(s + bias)
            r = 1.0 / jnp.sum(p, axis=1, keepdims=True)
            o = jnp.dot(p.astype(jnp.bfloat16), v_c[:, sl],
                        preferred_element_type=jnp.float32)
            outs.append(o * r)
        rows.append(jnp.concatenate(outs, axis=1))
    attn = jnp.concatenate(rows, axis=0).astype(jnp.bfloat16)
    a = jnp.dot(attn, wo_ref[...], preferred_element_type=jnp.float32)

    msk = isn_ref[0, pl.ds(t0, _BT), :] != 0
    o_ref[0] = jnp.where(msk, a, ec)


def kernel(embeds, is_numbers, Wq, Wk, Wv, Wo):
    B, T, D = embeds.shape
    dh = D // N_HEADS
    e_pad = jnp.pad(embeds, ((0, 0), (N_LEFT, N_RIGHT), (0, 0)))
    isn = is_numbers.astype(jnp.int32).reshape(B, T, 1)
    scale = 1.0 / (dh ** 0.5)
    w3 = jnp.concatenate([Wq * scale, Wk, Wv], axis=1).astype(jnp.bfloat16)
    wo = Wo.astype(jnp.bfloat16)
    W = N_LEFT + N_RIGHT
    return pl.pallas_call(
        functools.partial(_attn_body, T=T, D=D),
        grid=(B, T // _BT),
        in_specs=[
            pl.BlockSpec((1, T + W, D), lambda b, i: (b, 0, 0)),
            pl.BlockSpec((1, T, 1), lambda b, i: (b, 0, 0)),
            pl.BlockSpec((D, 3 * D), lambda b, i: (0, 0)),
            pl.BlockSpec((D, D), lambda b, i: (0, 0)),
        ],
        out_specs=pl.BlockSpec((1, _BT, D), lambda b, i: (b, i, 0)),
        out_shape=jax.ShapeDtypeStruct((B, T, D), jnp.float32),
        compiler_params=pltpu.CompilerParams(
            dimension_semantics=("parallel", "arbitrary"),
        ),
    )(e_pad, isn, w3, wo)
